# baseline (speedup 1.0000x reference)
import functools

import jax
import jax.numpy as jnp
from jax import lax
from jax.experimental import pallas as pl
from jax.experimental.pallas import tpu as pltpu
from jax.experimental.pallas import tpu_sc as plsc

_VOCAB = 1000000
_EMBED_DIM = 64
_BATCH = 4096
_SEQ_LEN = 200
_PAD_W = 128

_NC = 2
_NS = 16
_NW = _NC * _NS
_COLS_PER_W = _BATCH // _NW
_L = 16


def _transpose_block(gbuf, obuf):
    iot = lax.iota(jnp.int32, _L)
    rots = [(iot + k) % _L for k in range(_L)]
    rows = [b0 + iot for b0 in range(0, _COLS_PER_W, _L)]
    depth = 16

    def eblk(ei, carry):
        e0 = ei * _L
        pairs = []
        for k in range(_L):
            lcol = rots[k] + e0
            for lrow in rows:
                pairs.append((lrow, lcol))
        vals = {}
        for i, (lr, lc) in enumerate(pairs):
            vals[i] = plsc.load_gather(gbuf, [lr, lc])
            j = i - depth
            if j >= 0:
                lrj, lcj = pairs[j]
                plsc.store_scatter(obuf, [lcj, lrj], vals.pop(j))
        n = len(pairs)
        for j in range(n - depth, n):
            lrj, lcj = pairs[j]
            plsc.store_scatter(obuf, [lcj, lrj], vals.pop(j))
        return carry

    lax.fori_loop(0, _EMBED_DIM // _L, eblk, 0)


_NG = 3
_NO = 2
_PERIOD = 6


def _gather_pipeline(ids_hbm, table_hbm, out_hbm, idx_v, gbuf_v, obuf_v,
                     gsem, ssem):
    wid = lax.axis_index("s") * _NC + lax.axis_index("c")
    col0 = wid * _COLS_PER_W

    pltpu.sync_copy(ids_hbm.at[:, pl.ds(col0, _COLS_PER_W)], idx_v)

    def start_gather(s, b):
        pltpu.async_copy(table_hbm.at[idx_v.at[s]], gbuf_v.at[b], gsem.at[b])

    def wait_gather(b):
        pltpu.make_async_copy(
            table_hbm.at[idx_v.at[0]], gbuf_v.at[b], gsem.at[b]).wait()

    def start_store(s, b):
        pltpu.async_copy(
            obuf_v.at[b],
            out_hbm.at[pl.ds(s * _EMBED_DIM, _EMBED_DIM),
                       pl.ds(col0, _COLS_PER_W)],
            ssem.at[b])

    def wait_store(b):
        pltpu.make_async_copy(
            obuf_v.at[b],
            out_hbm.at[pl.ds(0, _EMBED_DIM), pl.ds(col0, _COLS_PER_W)],
            ssem.at[b]).wait()

    def step(s, has_store_wait=True, refill=True):
        g, o = s % _NG, s % _NO
        wait_gather(g)
        if has_store_wait:
            wait_store(o)
        _transpose_block(gbuf_v.at[g], obuf_v.at[o])
        start_store(s, o)
        if refill:
            start_gather(s + _NG, g)

    for s in range(_NG):
        start_gather(s, s)
    for s in range(_NG):
        step(s, has_store_wait=(s >= _NO))

    n_main = ((_SEQ_LEN - 2 * _NG) // _PERIOD) * _PERIOD
    def main_body(gidx, carry):
        for u in range(_PERIOD):
            step(_NG + gidx * _PERIOD + u)
        return carry
    lax.fori_loop(0, n_main // _PERIOD, main_body, 0)

    for s in range(_NG + n_main, _SEQ_LEN):
        step(s, refill=(s + _NG < _SEQ_LEN))
    for o in range(_NO):
        wait_store(o)


def kernel(input_ids, table):
    ids_t = input_ids.T.astype(jnp.int32)
    table_p = jnp.pad(table, ((0, 0), (0, _PAD_W - _EMBED_DIM)))

    mesh = plsc.VectorSubcoreMesh(core_axis_name="c", subcore_axis_name="s")
    gather = functools.partial(
        pl.kernel,
        mesh=mesh,
        out_type=jax.ShapeDtypeStruct((_SEQ_LEN * _EMBED_DIM, _BATCH),
                                      jnp.float32),
        scratch_types=[
            pltpu.VMEM((_SEQ_LEN, _COLS_PER_W), jnp.int32),
            pltpu.VMEM((_NG, _COLS_PER_W, _PAD_W), jnp.float32),
            pltpu.VMEM((_NO, _EMBED_DIM, _COLS_PER_W), jnp.float32),
            pltpu.SemaphoreType.DMA((_NG,)),
            pltpu.SemaphoreType.DMA((_NO,)),
        ],
        compiler_params=pltpu.CompilerParams(use_tc_tiling_on_sc=True,
                                             needs_layout_passes=False),
    )(_gather_pipeline)

    out = gather(ids_t, table_p)
    out = out.reshape(_SEQ_LEN, _EMBED_DIM, _BATCH)
    return out.transpose(2, 0, 1)


# speedup vs baseline: 1.0009x; 1.0009x over previous
import functools

import jax
import jax.numpy as jnp
from jax import lax
from jax.experimental import pallas as pl
from jax.experimental.pallas import tpu as pltpu
from jax.experimental.pallas import tpu_sc as plsc

_VOCAB = 1000000
_EMBED_DIM = 64
_BATCH = 4096
_SEQ_LEN = 200
_PAD_W = 128

_NC = 2
_NS = 16
_NW = _NC * _NS
_COLS_PER_W = _BATCH // _NW
_L = 16


def _transpose_block(gbuf, obuf):
    iot = lax.iota(jnp.int32, _L)
    rots = [(iot + k) % _L for k in range(_L)]
    rows = [b0 + iot for b0 in range(0, _COLS_PER_W, _L)]
    depth = 16

    def eblk(ei, carry):
        e0 = ei * _L
        pairs = []
        for k in range(_L):
            lcol = rots[k] + e0
            for lrow in rows:
                pairs.append((lrow, lcol))
        vals = {}
        for i, (lr, lc) in enumerate(pairs):
            vals[i] = plsc.load_gather(gbuf, [lr, lc])
            j = i - depth
            if j >= 0:
                lrj, lcj = pairs[j]
                plsc.store_scatter(obuf, [lcj, lrj], vals.pop(j))
        n = len(pairs)
        for j in range(n - depth, n):
            lrj, lcj = pairs[j]
            plsc.store_scatter(obuf, [lcj, lrj], vals.pop(j))
        return carry

    lax.fori_loop(0, _EMBED_DIM // _L, eblk, 0)


_NG = 3
_NO = 2
_PERIOD = 6


def _gather_pipeline(ids_hbm, table_hbm, out_hbm, idx_v, gbuf_v, obuf_v,
                     gsem, ssem):
    wid = lax.axis_index("s") * _NC + lax.axis_index("c")
    col0 = wid * _COLS_PER_W

    pltpu.sync_copy(ids_hbm.at[:, pl.ds(col0, _COLS_PER_W)], idx_v)

    def start_gather(s, b):
        pltpu.async_copy(table_hbm.at[idx_v.at[s]], gbuf_v.at[b], gsem.at[b])

    def wait_gather(b):
        pltpu.make_async_copy(
            table_hbm.at[idx_v.at[0]], gbuf_v.at[b], gsem.at[b]).wait()

    def start_store(s, b):
        pltpu.async_copy(
            obuf_v.at[b],
            out_hbm.at[pl.ds(s * _EMBED_DIM, _EMBED_DIM),
                       pl.ds(col0, _COLS_PER_W)],
            ssem.at[b])

    def wait_store(b):
        pltpu.make_async_copy(
            obuf_v.at[b],
            out_hbm.at[pl.ds(0, _EMBED_DIM), pl.ds(col0, _COLS_PER_W)],
            ssem.at[b]).wait()

    def step(s, has_store_wait=True, refill=True):
        g, o = s % _NG, s % _NO
        wait_gather(g)
        if has_store_wait:
            wait_store(o)
        _transpose_block(gbuf_v.at[g], obuf_v.at[o])
        start_store(s, o)
        if refill:
            start_gather(s + _NG, g)

    for s in range(_NG):
        start_gather(s, s)
    for s in range(_NG):
        step(s, has_store_wait=(s >= _NO))

    n_main = ((_SEQ_LEN - 2 * _NG) // _PERIOD) * _PERIOD
    def main_body(gidx, carry):
        for u in range(_PERIOD):
            step(_NG + gidx * _PERIOD + u)
        return carry
    lax.fori_loop(0, n_main // _PERIOD, main_body, 0)

    for s in range(_NG + n_main, _SEQ_LEN):
        step(s, refill=(s + _NG < _SEQ_LEN))
    for o in range(_NO):
        wait_store(o)


def kernel(input_ids, table):
    ids_t = input_ids.T.astype(jnp.int32)
    table_p = jnp.pad(table, ((0, 0), (0, _PAD_W - _EMBED_DIM)))

    mesh = plsc.VectorSubcoreMesh(core_axis_name="c", subcore_axis_name="s")
    gather = functools.partial(
        pl.kernel,
        mesh=mesh,
        out_type=jax.ShapeDtypeStruct((_SEQ_LEN * _EMBED_DIM, _BATCH),
                                      jnp.float32),
        scratch_types=[
            pltpu.VMEM((_SEQ_LEN, _COLS_PER_W), jnp.int32),
            pltpu.VMEM((_NG, _COLS_PER_W, _PAD_W), jnp.float32),
            pltpu.VMEM((_NO, _EMBED_DIM, _COLS_PER_W), jnp.float32),
            pltpu.SemaphoreType.DMA((_NG,)),
            pltpu.SemaphoreType.DMA((_NO,)),
        ],
        compiler_params=pltpu.CompilerParams(use_tc_tiling_on_sc=True,
                                             needs_layout_passes=False,
                                             skip_device_barrier=True),
    )(_gather_pipeline)

    out = gather(ids_t, table_p)
    out = out.reshape(_SEQ_LEN, _EMBED_DIM, _BATCH)
    return out.transpose(2, 0, 1)


# speedup vs baseline: 1.2983x; 1.2971x over previous
import functools

import jax
import jax.numpy as jnp
from jax import lax
from jax.experimental import pallas as pl
from jax.experimental.pallas import tpu as pltpu
from jax.experimental.pallas import tpu_sc as plsc

_VOCAB = 1000000
_EMBED_DIM = 64
_BATCH = 4096
_SEQ_LEN = 200
_PAD_W = 128

_NC = 2
_NS = 16
_NW = _NC * _NS
_COLS_PER_W = _BATCH // _NW
_L = 16


def _transpose_block(gbuf, obuf):
    iot = lax.iota(jnp.int32, _L)
    rots = [(iot + k) % _L for k in range(_L)]
    rows = [b0 + iot for b0 in range(0, _COLS_PER_W, _L)]
    depth = 16

    def eblk(ei, carry):
        e0 = ei * _L
        pairs = []
        for k in range(_L):
            lcol = rots[k] + e0
            for lrow in rows:
                pairs.append((lrow, lcol))
        vals = {}
        for i, (lr, lc) in enumerate(pairs):
            vals[i] = plsc.load_gather(gbuf, [lr, lc])
            j = i - depth
            if j >= 0:
                lrj, lcj = pairs[j]
                plsc.store_scatter(obuf, [lcj, lrj], vals.pop(j))
        n = len(pairs)
        for j in range(n - depth, n):
            lrj, lcj = pairs[j]
            plsc.store_scatter(obuf, [lcj, lrj], vals.pop(j))
        return carry

    lax.fori_loop(0, _EMBED_DIM // _L, eblk, 0)


_NG = 3
_NO = 2
_PERIOD = 6


def _gather_pipeline(ids_hbm, table_hbm, out_hbm, idx_v, gbuf_v, obuf_v,
                     gsem, ssem):
    wid = lax.axis_index("s") * _NC + lax.axis_index("c")
    col0 = wid * _COLS_PER_W

    pltpu.sync_copy(ids_hbm.at[:, pl.ds(col0, _COLS_PER_W)], idx_v)

    def start_gather(s, b):
        pltpu.async_copy(table_hbm.at[idx_v.at[s]], gbuf_v.at[b], gsem.at[b])

    def wait_gather(b):
        pltpu.make_async_copy(
            table_hbm.at[idx_v.at[0]], gbuf_v.at[b], gsem.at[b]).wait()

    def start_store(s, b):
        pltpu.async_copy(
            obuf_v.at[b],
            out_hbm.at[pl.ds(s * _EMBED_DIM, _EMBED_DIM),
                       pl.ds(col0, _COLS_PER_W)],
            ssem.at[b])

    def wait_store(b):
        pltpu.make_async_copy(
            obuf_v.at[b],
            out_hbm.at[pl.ds(0, _EMBED_DIM), pl.ds(col0, _COLS_PER_W)],
            ssem.at[b]).wait()

    def step(s, has_store_wait=True, refill=True):
        g, o = s % _NG, s % _NO
        wait_gather(g)
        if has_store_wait:
            wait_store(o)
        _transpose_block(gbuf_v.at[g], obuf_v.at[o])
        start_store(s, o)
        if refill:
            start_gather(s + _NG, g)

    for s in range(_NG):
        start_gather(s, s)
    for s in range(_NG):
        step(s, has_store_wait=(s >= _NO))

    n_main = ((_SEQ_LEN - 2 * _NG) // _PERIOD) * _PERIOD
    def main_body(gidx, carry):
        for u in range(_PERIOD):
            step(_NG + gidx * _PERIOD + u)
        return carry
    lax.fori_loop(0, n_main // _PERIOD, main_body, 0)

    for s in range(_NG + n_main, _SEQ_LEN):
        step(s, refill=(s + _NG < _SEQ_LEN))
    for o in range(_NO):
        wait_store(o)


_VPAD = 1000064
_NBLK = _VPAD // 128


def _relayout_pipeline(tt_hbm, p_hbm, ibuf_v, tbuf_v, rsem, wsem):
    wid = lax.axis_index("s") * _NC + lax.axis_index("c")
    nblk = 244 + jnp.where(wid < _NBLK - 244 * _NW, 1, 0)

    iot = lax.iota(jnp.int32, _L)
    rots = [(iot + k) % _L for k in range(_L)]
    erows = [e0 + iot for e0 in range(0, _EMBED_DIM, _L)]

    def transpose(ib, tb):
        def vblk(vbi, carry):
            vb0 = vbi * _L
            pairs = []
            for k in range(_L):
                vcol = rots[k] + vb0
                for er in erows:
                    pairs.append((er, vcol))
            vals = {}
            for i, (er, vc) in enumerate(pairs):
                vals[i] = plsc.load_gather(ib, [er, vc])
                j = i - 16
                if j >= 0:
                    erj, vcj = pairs[j]
                    plsc.store_scatter(tb, [vcj, erj], vals.pop(j))
            n = len(pairs)
            for j in range(n - 16, n):
                erj, vcj = pairs[j]
                plsc.store_scatter(tb, [vcj, erj], vals.pop(j))
            return carry
        lax.fori_loop(0, 128 // _L, vblk, 0)

    def start_read(i, b):
        v0 = (wid + _NW * i) * 128
        pltpu.async_copy(tt_hbm.at[:, pl.ds(v0, 128)], ibuf_v.at[b],
                         rsem.at[b])

    def wait_read(b):
        pltpu.make_async_copy(
            tt_hbm.at[:, pl.ds(0, 128)], ibuf_v.at[b], rsem.at[b]).wait()

    def start_write(i, b):
        v0 = (wid + _NW * i) * 128
        pltpu.async_copy(tbuf_v.at[b], p_hbm.at[pl.ds(v0, 128)], wsem.at[b])

    def wait_write(b):
        pltpu.make_async_copy(
            tbuf_v.at[b], p_hbm.at[pl.ds(0, 128)], wsem.at[b]).wait()

    start_read(0, 0)

    def step(i):
        b = i % 2

        @pl.when(i < nblk)
        def _():
            wait_read(b)

            @pl.when(i + 1 < nblk)
            def _():
                start_read(i + 1, 1 - b)

            @pl.when(i >= 2)
            def _():
                wait_write(b)

            transpose(ibuf_v.at[b], tbuf_v.at[b])
            start_write(i, b)

    step(0)
    def main_body(g, carry):
        step(1 + 2 * g)
        step(2 + 2 * g)
        return carry
    lax.fori_loop(0, 122, main_body, 0)
    for b in range(2):
        wait_write(b)


def kernel(input_ids, table):
    ids_t = input_ids.T.astype(jnp.int32)
    tt = table.T

    mesh = plsc.VectorSubcoreMesh(core_axis_name="c", subcore_axis_name="s")
    relayout = functools.partial(
        pl.kernel,
        mesh=mesh,
        out_type=jax.ShapeDtypeStruct((_VPAD, _PAD_W), jnp.float32),
        scratch_types=[
            pltpu.VMEM((2, _EMBED_DIM, 128), jnp.float32),
            pltpu.VMEM((2, 128, _PAD_W), jnp.float32),
            pltpu.SemaphoreType.DMA((2,)),
            pltpu.SemaphoreType.DMA((2,)),
        ],
        compiler_params=pltpu.CompilerParams(use_tc_tiling_on_sc=True,
                                             needs_layout_passes=False,
                                             disable_bounds_checks=True),
    )(_relayout_pipeline)
    table_p = relayout(tt)
    gather = functools.partial(
        pl.kernel,
        mesh=mesh,
        out_type=jax.ShapeDtypeStruct((_SEQ_LEN * _EMBED_DIM, _BATCH),
                                      jnp.float32),
        scratch_types=[
            pltpu.VMEM((_SEQ_LEN, _COLS_PER_W), jnp.int32),
            pltpu.VMEM((_NG, _COLS_PER_W, _PAD_W), jnp.float32),
            pltpu.VMEM((_NO, _EMBED_DIM, _COLS_PER_W), jnp.float32),
            pltpu.SemaphoreType.DMA((_NG,)),
            pltpu.SemaphoreType.DMA((_NO,)),
        ],
        compiler_params=pltpu.CompilerParams(use_tc_tiling_on_sc=True,
                                             needs_layout_passes=False),
    )(_gather_pipeline)

    out = gather(ids_t, table_p)
    out = out.reshape(_SEQ_LEN, _EMBED_DIM, _BATCH)
    return out.transpose(2, 0, 1)
